# baseline (speedup 1.0000x reference)
import jax
import jax.numpy as jnp
from jax.experimental import pallas as pl
from jax.experimental.pallas import tpu as pltpu

_TR = 1000


def _msg_add_kernel(msg_ref, even_ref, odd_ref, hid_ref, out_ref):
    j = pl.program_id(0)
    nbits = even_ref.shape[0]
    b = j // (8000 // _TR)
    even = even_ref[...]
    diff = odd_ref[...] - even
    acc = jnp.sum(even, axis=0, keepdims=True)
    for i in range(nbits):
        bit = msg_ref[b, i]
        acc = acc + bit * diff[i : i + 1, :]
    out_ref[...] = hid_ref[...] + acc


def kernel(hidden, msg, emb):
    B, H, T = hidden.shape
    nbits = msg.shape[-1]
    even = emb[0::2].astype(jnp.float32)
    odd = emb[1::2].astype(jnp.float32)
    msg_f = msg.astype(jnp.float32)
    hid2 = hidden.transpose(0, 2, 1).reshape(B * T, H)
    grid = (B * T // _TR,)
    out = pl.pallas_call(
        _msg_add_kernel,
        grid=grid,
        in_specs=[
            pl.BlockSpec(memory_space=pltpu.SMEM),
            pl.BlockSpec((nbits, H), lambda j: (0, 0)),
            pl.BlockSpec((nbits, H), lambda j: (0, 0)),
            pl.BlockSpec((_TR, H), lambda j: (j, 0)),
        ],
        out_specs=pl.BlockSpec((_TR, H), lambda j: (j, 0)),
        out_shape=jax.ShapeDtypeStruct((B * T, H), jnp.float32),
        compiler_params=pltpu.CompilerParams(
            dimension_semantics=("parallel",),
        ),
    )(msg_f, even, odd, hid2)
    return out.reshape(B, T, H).transpose(0, 2, 1)


# speedup vs baseline: 1.1139x; 1.1139x over previous
import jax
import jax.numpy as jnp
from jax.experimental import pallas as pl
from jax.experimental.pallas import tpu as pltpu

_TR = 2000
_NBUF = 4
_T = 8000


def _outer(msg_ref, even_ref, odd_ref, hid_hbm, out_hbm):
    nbits = even_ref.shape[0]
    blocks_per_batch = _T // _TR

    def body(idxs, hid_blk, out_blk):
        j = idxs[0]
        b = j // blocks_per_batch
        even = even_ref[...]
        diff = odd_ref[...] - even
        acc = jnp.sum(even, axis=0, keepdims=True)
        for i in range(nbits):
            bit = msg_ref[b, i]
            acc = acc + bit * diff[i : i + 1, :]
        out_blk[...] = hid_blk[...] + acc

    nblk = hid_hbm.shape[0] // _TR
    H = hid_hbm.shape[1]
    pltpu.emit_pipeline(
        body,
        grid=(nblk,),
        in_specs=[
            pl.BlockSpec((_TR, H), lambda j: (j, 0),
                         pipeline_mode=pl.Buffered(buffer_count=_NBUF)),
        ],
        out_specs=[
            pl.BlockSpec((_TR, H), lambda j: (j, 0),
                         pipeline_mode=pl.Buffered(buffer_count=2)),
        ],
        _explicit_indices=True,
    )(hid_hbm, out_hbm)


def kernel(hidden, msg, emb):
    B, H, T = hidden.shape
    nbits = msg.shape[-1]
    even = emb[0::2].astype(jnp.float32)
    odd = emb[1::2].astype(jnp.float32)
    msg_f = msg.astype(jnp.float32)
    hid2 = hidden.transpose(0, 2, 1).reshape(B * T, H)
    out = pl.pallas_call(
        _outer,
        in_specs=[
            pl.BlockSpec(memory_space=pltpu.SMEM),
            pl.BlockSpec(memory_space=pltpu.VMEM),
            pl.BlockSpec(memory_space=pltpu.VMEM),
            pl.BlockSpec(memory_space=pl.ANY),
        ],
        out_specs=pl.BlockSpec(memory_space=pl.ANY),
        out_shape=jax.ShapeDtypeStruct((B * T, H), jnp.float32),
    )(msg_f, even, odd, hid2)
    return out.reshape(B, T, H).transpose(0, 2, 1)


# speedup vs baseline: 1.1160x; 1.0019x over previous
import jax
import jax.numpy as jnp
from jax.experimental import pallas as pl
from jax.experimental.pallas import tpu as pltpu

_TR = 4000
_NBUF = 3
_T = 8000


def _outer(msg_ref, even_ref, odd_ref, hid_hbm, out_hbm):
    nbits = even_ref.shape[0]
    blocks_per_batch = _T // _TR

    def body(idxs, hid_blk, out_blk):
        j = idxs[0]
        b = j // blocks_per_batch
        even = even_ref[...]
        diff = odd_ref[...] - even
        acc = jnp.sum(even, axis=0, keepdims=True)
        for i in range(nbits):
            bit = msg_ref[b, i]
            acc = acc + bit * diff[i : i + 1, :]
        out_blk[...] = hid_blk[...] + acc

    nblk = hid_hbm.shape[0] // _TR
    H = hid_hbm.shape[1]
    pltpu.emit_pipeline(
        body,
        grid=(nblk,),
        in_specs=[
            pl.BlockSpec((_TR, H), lambda j: (j, 0),
                         pipeline_mode=pl.Buffered(buffer_count=_NBUF)),
        ],
        out_specs=[
            pl.BlockSpec((_TR, H), lambda j: (j, 0),
                         pipeline_mode=pl.Buffered(buffer_count=2)),
        ],
        _explicit_indices=True,
    )(hid_hbm, out_hbm)


def kernel(hidden, msg, emb):
    B, H, T = hidden.shape
    nbits = msg.shape[-1]
    even = emb[0::2].astype(jnp.float32)
    odd = emb[1::2].astype(jnp.float32)
    msg_f = msg.astype(jnp.float32)
    hid2 = hidden.transpose(0, 2, 1).reshape(B * T, H)
    out = pl.pallas_call(
        _outer,
        in_specs=[
            pl.BlockSpec(memory_space=pltpu.SMEM),
            pl.BlockSpec(memory_space=pltpu.VMEM),
            pl.BlockSpec(memory_space=pltpu.VMEM),
            pl.BlockSpec(memory_space=pl.ANY),
        ],
        out_specs=pl.BlockSpec(memory_space=pl.ANY),
        out_shape=jax.ShapeDtypeStruct((B * T, H), jnp.float32),
    )(msg_f, even, odd, hid2)
    return out.reshape(B, T, H).transpose(0, 2, 1)


# speedup vs baseline: 1.1161x; 1.0001x over previous
import jax
import jax.numpy as jnp
from jax.experimental import pallas as pl
from jax.experimental.pallas import tpu as pltpu

_TR = 4000


def _msg_add_kernel(msg_ref, even_ref, odd_ref, hid_ref, out_ref, acc_ref):
    j = pl.program_id(0)
    nbits = even_ref.shape[0]
    bpb = 8000 // _TR
    b = j // bpb

    @pl.when(j % bpb == 0)
    def _():
        even = even_ref[...]
        diff = odd_ref[...] - even
        acc = jnp.sum(even, axis=0, keepdims=True)
        for i in range(nbits):
            bit = msg_ref[b, i]
            acc = acc + bit * diff[i : i + 1, :]
        acc_ref[...] = acc

    out_ref[...] = hid_ref[...] + acc_ref[...]


def kernel(hidden, msg, emb):
    B, H, T = hidden.shape
    nbits = msg.shape[-1]
    even = emb[0::2].astype(jnp.float32)
    odd = emb[1::2].astype(jnp.float32)
    msg_f = msg.astype(jnp.float32)
    hid2 = hidden.transpose(0, 2, 1).reshape(B * T, H)
    grid = (B * T // _TR,)
    out = pl.pallas_call(
        _msg_add_kernel,
        grid=grid,
        in_specs=[
            pl.BlockSpec(memory_space=pltpu.SMEM),
            pl.BlockSpec((nbits, H), lambda j: (0, 0)),
            pl.BlockSpec((nbits, H), lambda j: (0, 0)),
            pl.BlockSpec((_TR, H), lambda j: (j, 0)),
        ],
        out_specs=pl.BlockSpec((_TR, H), lambda j: (j, 0)),
        out_shape=jax.ShapeDtypeStruct((B * T, H), jnp.float32),
        scratch_shapes=[pltpu.VMEM((1, H), jnp.float32)],
        compiler_params=pltpu.CompilerParams(
            dimension_semantics=("arbitrary",),
        ),
    )(msg_f, even, odd, hid2)
    return out.reshape(B, T, H).transpose(0, 2, 1)
